# Initial kernel scaffold; baseline (speedup 1.0000x reference)
import jax
import jax.numpy as jnp
from jax.experimental import pallas as pl


def kernel(inputs, pos1, pos2, word_table, pos1_table, pos2_table):
    raise NotImplementedError("write your pallas kernel here")


# speedup vs baseline: 4.7633x; 4.7633x over previous
import functools

import jax
import jax.numpy as jnp
from jax import lax
from jax.experimental import pallas as pl
from jax.experimental.pallas import tpu as pltpu
from jax.experimental.pallas import tpu_sc as plsc

BATCH = 4096
SEQ = 200
WORD_D = 64
POS_D = 32
OUT_D = WORD_D + 2 * POS_D
N_TOK = BATCH * SEQ

_INFO = plsc.get_sparse_core_info()
NC = _INFO.num_cores
NS = _INFO.num_subcores
NW = NC * NS
L = _INFO.num_lanes

TPW = N_TOK // NW
IDX_W = 128
CHUNK = 512
ROWS = CHUNK // IDX_W
N_CHUNK = TPW // CHUNK
ROWS_PW = TPW // IDX_W


def _embed_body(w_idx, p1_idx, p2_idx, wtab, p1tab, p2tab, out,
                widx_v, p1idx_v, p2idx_v, wrows, p1rows, p2rows, sem):
    c = lax.axis_index("c")
    s = lax.axis_index("s")
    wid = s * NC + c

    def chunk_body(i, carry):
        row0 = wid * ROWS_PW + i * ROWS
        base = row0 * IDX_W
        pltpu.sync_copy(w_idx.at[pl.ds(row0, ROWS)], widx_v)
        pltpu.sync_copy(p1_idx.at[pl.ds(row0, ROWS)], p1idx_v)
        pltpu.sync_copy(p2_idx.at[pl.ds(row0, ROWS)], p2idx_v)
        handles = []
        for j in range(ROWS):
            handles.append(pltpu.async_copy(
                wtab.at[widx_v.at[j]], wrows.at[pl.ds(j * IDX_W, IDX_W)], sem))
            handles.append(pltpu.async_copy(
                p1tab.at[p1idx_v.at[j]], p1rows.at[pl.ds(j * IDX_W, IDX_W)], sem))
            handles.append(pltpu.async_copy(
                p2tab.at[p2idx_v.at[j]], p2rows.at[pl.ds(j * IDX_W, IDX_W)], sem))
        for h in handles:
            h.wait()
        pltpu.sync_copy(wrows, out.at[pl.ds(base, CHUNK), pl.ds(0, WORD_D)])
        pltpu.sync_copy(p1rows, out.at[pl.ds(base, CHUNK), pl.ds(WORD_D, POS_D)])
        pltpu.sync_copy(p2rows, out.at[pl.ds(base, CHUNK), pl.ds(WORD_D + POS_D, POS_D)])
        return carry

    lax.fori_loop(0, N_CHUNK, chunk_body, 0)


@functools.partial(
    pl.kernel,
    out_type=jax.ShapeDtypeStruct((N_TOK, OUT_D), jnp.float32),
    mesh=plsc.VectorSubcoreMesh(core_axis_name="c", subcore_axis_name="s"),
    compiler_params=pltpu.CompilerParams(use_tc_tiling_on_sc=False),
    scratch_types=[
        pltpu.VMEM((ROWS, IDX_W), jnp.int32),
        pltpu.VMEM((ROWS, IDX_W), jnp.int32),
        pltpu.VMEM((ROWS, IDX_W), jnp.int32),
        pltpu.VMEM((CHUNK, WORD_D), jnp.float32),
        pltpu.VMEM((CHUNK, POS_D), jnp.float32),
        pltpu.VMEM((CHUNK, POS_D), jnp.float32),
        pltpu.SemaphoreType.DMA,
    ],
)
def _embed(*args):
    _embed_body(*args)


def kernel(inputs, pos1, pos2, word_table, pos1_table, pos2_table):
    w_idx = inputs.reshape(N_TOK // IDX_W, IDX_W)
    p1_idx = pos1.reshape(N_TOK // IDX_W, IDX_W)
    p2_idx = pos2.reshape(N_TOK // IDX_W, IDX_W)
    out = _embed(w_idx, p1_idx, p2_idx, word_table, pos1_table, pos2_table)
    return out.reshape(BATCH, SEQ, OUT_D)
